# baseline (speedup 1.0000x reference)
import jax
import jax.numpy as jnp
from jax.experimental import pallas as pl
from jax.experimental.pallas import tpu as pltpu

_B, _H, _Q, _D, _S = 8, 16, 16, 64, 2048


def _kv_update_kernel(pos_ref, k_val_ref, v_val_ref, k_cache_ref, v_cache_ref,
                      k_out_ref, v_out_ref):
    b = pl.program_id(0)
    k_out_ref[...] = k_cache_ref[...]
    v_out_ref[...] = v_cache_ref[...]
    rows = jax.lax.broadcasted_iota(jnp.int32, (8, _D), 0)
    for q in range(_Q):
        p = pos_ref[b, q]
        base = (p // 8) * 8
        mask = rows == (p - base)
        kv = jnp.broadcast_to(k_val_ref[0, 0, q, :][None, :], (8, _D))
        vv = jnp.broadcast_to(v_val_ref[0, 0, q, :][None, :], (8, _D))
        kwin = k_out_ref[0, 0, pl.ds(base, 8), :]
        vwin = v_out_ref[0, 0, pl.ds(base, 8), :]
        k_out_ref[0, 0, pl.ds(base, 8), :] = jnp.where(mask, kv, kwin)
        v_out_ref[0, 0, pl.ds(base, 8), :] = jnp.where(mask, vv, vwin)


def kernel(input_pos, k_val, v_val, k_cache, v_cache):
    grid_spec = pltpu.PrefetchScalarGridSpec(
        num_scalar_prefetch=1,
        grid=(_B, _H),
        in_specs=[
            pl.BlockSpec((1, 1, _Q, _D), lambda b, h, pos: (b, h, 0, 0)),
            pl.BlockSpec((1, 1, _Q, _D), lambda b, h, pos: (b, h, 0, 0)),
            pl.BlockSpec((1, 1, _S, _D), lambda b, h, pos: (b, h, 0, 0)),
            pl.BlockSpec((1, 1, _S, _D), lambda b, h, pos: (b, h, 0, 0)),
        ],
        out_specs=[
            pl.BlockSpec((1, 1, _S, _D), lambda b, h, pos: (b, h, 0, 0)),
            pl.BlockSpec((1, 1, _S, _D), lambda b, h, pos: (b, h, 0, 0)),
        ],
    )
    return pl.pallas_call(
        _kv_update_kernel,
        grid_spec=grid_spec,
        out_shape=[
            jax.ShapeDtypeStruct(k_cache.shape, k_cache.dtype),
            jax.ShapeDtypeStruct(v_cache.shape, v_cache.dtype),
        ],
    )(input_pos, k_val, v_val, k_cache, v_cache)
